# baseline (speedup 1.0000x reference)
import functools

import jax
import jax.numpy as jnp
from jax import lax
from jax.experimental import pallas as pl
from jax.experimental.pallas import tpu as pltpu
from jax.experimental.pallas import tpu_sc as plsc

N_NODES = 16384
EXPR_LEN = 50
IDENT_DIM = 128
HALF = IDENT_DIM // 2
CONTROL_VOCAB = 32
CONTROL_DIM = 8
OUT_DIM = IDENT_DIM + CONTROL_DIM
N_IDENT = 100000

GROUP = 4
NBUF = 4
FLUSH_GROUPS = 32
PACK_CHUNK = 125
PACK_NBUF = 4


def _sc_mesh_info():
    info = plsc.get_sparse_core_info()
    return info.num_cores, info.num_subcores, info.num_lanes


def _sc_pack_table(table_f32):
    nc, ns, nlanes = _sc_mesh_info()
    nw = nc * ns
    rpw = N_IDENT // nw
    nsteps = rpw // PACK_CHUNK

    mesh = plsc.VectorSubcoreMesh(core_axis_name="c", subcore_axis_name="s")

    @functools.partial(
        pl.kernel,
        mesh=mesh,
        compiler_params=pltpu.CompilerParams(
            needs_layout_passes=False, use_tc_tiling_on_sc=False),
        out_type=jax.ShapeDtypeStruct((N_IDENT, HALF), jnp.int32),
        scratch_types=[
            pltpu.VMEM((PACK_NBUF, PACK_CHUNK, IDENT_DIM), jnp.float32),
            pltpu.VMEM((PACK_NBUF, PACK_CHUNK, HALF), jnp.int32),
        ]
        + [pltpu.SemaphoreType.DMA for _ in range(2 * PACK_NBUF)],
    )
    def body(tab_hbm, out_hbm, in_v, out_v, *sems):
        wid = lax.axis_index("s") * nc + lax.axis_index("c")
        base = wid * rpw
        rnd = jnp.int32(0x8000)
        hi_mask = jnp.int32(-65536)
        sin = sems[:PACK_NBUF]
        sout = sems[PACK_NBUF:]

        def start_in(s, b):
            pltpu.async_copy(
                tab_hbm.at[pl.ds(base + s * PACK_CHUNK, PACK_CHUNK)],
                in_v.at[b], sin[b])

        def wait_in(b):
            pltpu.make_async_copy(
                tab_hbm.at[pl.ds(base, PACK_CHUNK)], in_v.at[b],
                sin[b]).wait()

        def start_out(s, b):
            pltpu.async_copy(
                out_v.at[b],
                out_hbm.at[pl.ds(base + s * PACK_CHUNK, PACK_CHUNK)],
                sout[b])

        def wait_out(b):
            pltpu.make_async_copy(
                out_v.at[b], out_hbm.at[pl.ds(base, PACK_CHUNK)],
                sout[b]).wait()

        for b in range(PACK_NBUF):
            start_in(b, b)

        def compute(b):
            @plsc.parallel_loop(0, PACK_CHUNK, unroll=10)
            def _rows(r):
                for c in range(HALF // 16):
                    a = in_v[b, r, pl.ds(16 * c, 16)]
                    bv = in_v[b, r, pl.ds(HALF + 16 * c, 16)]
                    pk = plsc.pack(a, bv, format=plsc.PackFormat.INTERLEAVED)
                    out_v[b, r, pl.ds(16 * c, 16)] = plsc.bitcast(
                        pk, jnp.int32)

        def step(s2, carry):
            for b in range(PACK_NBUF):
                s = s2 * PACK_NBUF + b
                wait_in(b)

                @pl.when(s2 >= 1)
                def _drain_out():
                    wait_out(b)

                compute(b)
                start_out(s, b)

                @pl.when(s + PACK_NBUF < nsteps)
                def _prefetch():
                    start_in(s + PACK_NBUF, b)

            return carry

        lax.fori_loop(0, nsteps // PACK_NBUF, step, 0, unroll=False)
        for t in range(nsteps - nsteps % PACK_NBUF, nsteps):
            b = t % PACK_NBUF
            wait_in(b)
            wait_out(b)
            compute(b)
            start_out(t, b)
        for b in range(PACK_NBUF):
            wait_out(b)

    return body(table_f32)


def _sc_pool_sum(flat_idx, packed_table):
    nc, ns, nlanes = _sc_mesh_info()
    nw = nc * ns
    npw = N_NODES // nw
    ngroups = npw // GROUP
    rows = GROUP * EXPR_LEN
    ipw = npw * EXPR_LEN
    out_rows = FLUSH_GROUPS * GROUP
    nchunks = HALF // nlanes

    mesh = plsc.VectorSubcoreMesh(core_axis_name="c", subcore_axis_name="s")

    @functools.partial(
        pl.kernel,
        mesh=mesh,
        compiler_params=pltpu.CompilerParams(
            needs_layout_passes=False, use_tc_tiling_on_sc=False),
        out_type=jax.ShapeDtypeStruct((N_NODES, IDENT_DIM), jnp.float32),
        scratch_types=[
            pltpu.VMEM((ipw,), jnp.int32),
            pltpu.VMEM((NBUF, rows, HALF), jnp.int32),
            pltpu.VMEM((out_rows, IDENT_DIM), jnp.float32),
        ]
        + [pltpu.SemaphoreType.DMA for _ in range(2 * NBUF)],
    )
    def body(idx_hbm, table_hbm, out_hbm, idx_v, ring_v, out_v, *sems):
        wid = lax.axis_index("s") * nc + lax.axis_index("c")
        nbase = wid * npw
        pltpu.sync_copy(idx_hbm.at[pl.ds(wid * ipw, ipw)], idx_v)

        split = 104

        def start(g, b):
            off = g * rows
            pltpu.async_copy(
                table_hbm.at[idx_v.at[pl.ds(off, split)]],
                ring_v.at[b, pl.ds(0, split)], sems[2 * b])
            pltpu.async_copy(
                table_hbm.at[idx_v.at[pl.ds(off + split, rows - split)]],
                ring_v.at[b, pl.ds(split, rows - split)], sems[2 * b + 1])

        def wait(b):
            pltpu.make_async_copy(
                table_hbm.at[idx_v.at[pl.ds(0, split)]],
                ring_v.at[b, pl.ds(0, split)], sems[2 * b]).wait()
            pltpu.make_async_copy(
                table_hbm.at[idx_v.at[pl.ds(0, rows - split)]],
                ring_v.at[b, pl.ds(split, rows - split)], sems[2 * b + 1]).wait()

        for b in range(NBUF):
            start(b, b)

        hi_mask = jnp.int32(-65536)

        def outer(gg, carry):
            g0 = gg * NBUF
            kk = gg // (FLUSH_GROUPS // NBUF)
            for b in range(NBUF):
                g = g0 + b
                wait(b)
                for h in range(GROUP):
                    row0 = h * EXPR_LEN
                    orow = (g - kk * FLUSH_GROUPS) * GROUP + h

                    zero = jnp.zeros((nlanes,), jnp.float32)

                    @plsc.parallel_loop(
                        0, EXPR_LEN, unroll=1,
                        carry=(zero,) * (2 * nchunks))
                    def accs(r, acc_in):
                        out = []
                        for c in range(nchunks):
                            w = ring_v[b, row0 + r, pl.ds(16 * c, 16)]
                            lo = lax.bitcast_convert_type(
                                lax.shift_left(w, 16), jnp.float32)
                            hi = lax.bitcast_convert_type(w, jnp.float32)
                            out.append(acc_in[2 * c] + lo)
                            out.append(acc_in[2 * c + 1] + hi)
                        return tuple(out)
                    for c in range(nchunks):
                        out_v[orow, pl.ds(16 * c, 16)] = accs[2 * c]
                        out_v[orow, pl.ds(HALF + 16 * c, 16)] = accs[2 * c + 1]

                    if h == GROUP - 1:
                        @pl.when(g + NBUF < ngroups)
                        def _start_next():
                            start(g + NBUF, b)

            @pl.when(gg % (FLUSH_GROUPS // NBUF) == FLUSH_GROUPS // NBUF - 1)
            def _flush():
                pltpu.sync_copy(
                    out_v,
                    out_hbm.at[pl.ds(nbase + kk * out_rows, out_rows)])

            return carry

        lax.fori_loop(0, ngroups // NBUF, outer, 0, unroll=False)

    return body(flat_idx, packed_table)


def _tc_project(pooled_sum, W_expr, b_expr, control_kind, control_kind_table):
    bn = 2048
    grid = (N_NODES // bn,)

    def body(x_ref, w_ref, b_ref, ck_ref, ctab_ref, o_ref):
        x = x_ref[...] * (1.0 / EXPR_LEN)
        y = jnp.dot(x, w_ref[...], preferred_element_type=jnp.float32)
        y = y + b_ref[...]
        ck = ck_ref[...]
        onehot = (
            ck == lax.broadcasted_iota(jnp.int32, (bn, CONTROL_VOCAB), 1)
        ).astype(jnp.float32)
        ctl = jnp.dot(onehot, ctab_ref[...], preferred_element_type=jnp.float32)
        o_ref[...] = jnp.concatenate([y, ctl], axis=-1)

    return pl.pallas_call(
        body,
        grid=grid,
        in_specs=[
            pl.BlockSpec((bn, IDENT_DIM), lambda i: (i, 0)),
            pl.BlockSpec((IDENT_DIM, IDENT_DIM), lambda i: (0, 0)),
            pl.BlockSpec((1, IDENT_DIM), lambda i: (0, 0)),
            pl.BlockSpec((bn, 1), lambda i: (i, 0)),
            pl.BlockSpec((CONTROL_VOCAB, CONTROL_DIM), lambda i: (0, 0)),
        ],
        out_specs=pl.BlockSpec((bn, OUT_DIM), lambda i: (i, 0)),
        out_shape=jax.ShapeDtypeStruct((N_NODES, OUT_DIM), jnp.float32),
    )(pooled_sum, W_expr, b_expr, control_kind, control_kind_table)


def kernel(encoded_identifiers, cfg_nodes_expressions, cfg_nodes_control_kind,
           W_expr, b_expr, control_kind_table):
    flat_idx = cfg_nodes_expressions.astype(jnp.int32).reshape(-1)
    ck = cfg_nodes_control_kind.astype(jnp.int32).reshape(N_NODES, 1)

    packed = _sc_pack_table(encoded_identifiers)
    pooled_sum = _sc_pool_sum(flat_idx, packed)
    return _tc_project(
        pooled_sum,
        W_expr,
        b_expr.reshape(1, IDENT_DIM),
        ck,
        control_kind_table,
    )


# speedup vs baseline: 1.1617x; 1.1617x over previous
import functools

import jax
import jax.numpy as jnp
from jax import lax
from jax.experimental import pallas as pl
from jax.experimental.pallas import tpu as pltpu
from jax.experimental.pallas import tpu_sc as plsc

N_NODES = 16384
EXPR_LEN = 50
IDENT_DIM = 128
HALF = IDENT_DIM // 2
CONTROL_VOCAB = 32
CONTROL_DIM = 8
OUT_DIM = IDENT_DIM + CONTROL_DIM
N_IDENT = 100000

GROUP = 4
NBUF = 4
FLUSH_GROUPS = 32
PACK_CHUNK = 125
PACK_NBUF = 4


def _sc_mesh_info():
    info = plsc.get_sparse_core_info()
    return info.num_cores, info.num_subcores, info.num_lanes


def _sc_pack_table(table_f32):
    nc, ns, nlanes = _sc_mesh_info()
    nw = nc * ns
    rpw = N_IDENT // nw
    nsteps = rpw // PACK_CHUNK

    mesh = plsc.VectorSubcoreMesh(core_axis_name="c", subcore_axis_name="s")

    @functools.partial(
        pl.kernel,
        mesh=mesh,
        compiler_params=pltpu.CompilerParams(
            needs_layout_passes=False, use_tc_tiling_on_sc=False),
        out_type=jax.ShapeDtypeStruct((N_IDENT, HALF), jnp.int32),
        scratch_types=[
            pltpu.VMEM((PACK_NBUF, PACK_CHUNK, IDENT_DIM), jnp.float32),
            pltpu.VMEM((PACK_NBUF, PACK_CHUNK, HALF), jnp.int32),
        ]
        + [pltpu.SemaphoreType.DMA for _ in range(2 * PACK_NBUF)],
    )
    def body(tab_hbm, out_hbm, in_v, out_v, *sems):
        wid = lax.axis_index("s") * nc + lax.axis_index("c")
        base = wid * rpw
        rnd = jnp.int32(0x8000)
        hi_mask = jnp.int32(-65536)
        sin = sems[:PACK_NBUF]
        sout = sems[PACK_NBUF:]

        def start_in(s, b):
            pltpu.async_copy(
                tab_hbm.at[pl.ds(base + s * PACK_CHUNK, PACK_CHUNK)],
                in_v.at[b], sin[b])

        def wait_in(b):
            pltpu.make_async_copy(
                tab_hbm.at[pl.ds(base, PACK_CHUNK)], in_v.at[b],
                sin[b]).wait()

        def start_out(s, b):
            pltpu.async_copy(
                out_v.at[b],
                out_hbm.at[pl.ds(base + s * PACK_CHUNK, PACK_CHUNK)],
                sout[b])

        def wait_out(b):
            pltpu.make_async_copy(
                out_v.at[b], out_hbm.at[pl.ds(base, PACK_CHUNK)],
                sout[b]).wait()

        for b in range(PACK_NBUF):
            start_in(b, b)

        def compute(b):
            @plsc.parallel_loop(0, PACK_CHUNK, unroll=10)
            def _rows(r):
                for c in range(HALF // 16):
                    a = in_v[b, r, pl.ds(16 * c, 16)]
                    bv = in_v[b, r, pl.ds(HALF + 16 * c, 16)]
                    pk = plsc.pack(a, bv, format=plsc.PackFormat.INTERLEAVED)
                    out_v[b, r, pl.ds(16 * c, 16)] = plsc.bitcast(
                        pk, jnp.int32)

        def step(s2, carry):
            for b in range(PACK_NBUF):
                s = s2 * PACK_NBUF + b
                wait_in(b)

                @pl.when(s2 >= 1)
                def _drain_out():
                    wait_out(b)

                compute(b)
                start_out(s, b)

                @pl.when(s + PACK_NBUF < nsteps)
                def _prefetch():
                    start_in(s + PACK_NBUF, b)

            return carry

        lax.fori_loop(0, nsteps // PACK_NBUF, step, 0, unroll=False)
        for t in range(nsteps - nsteps % PACK_NBUF, nsteps):
            b = t % PACK_NBUF
            wait_in(b)
            wait_out(b)
            compute(b)
            start_out(t, b)
        for b in range(PACK_NBUF):
            wait_out(b)

    return body(table_f32)


def _sc_pool_sum(flat_idx, packed_table):
    nc, ns, nlanes = _sc_mesh_info()
    nw = nc * ns
    npw = N_NODES // nw
    ngroups = npw // GROUP
    rows = GROUP * EXPR_LEN
    ipw = npw * EXPR_LEN
    out_rows = FLUSH_GROUPS * GROUP
    nchunks = HALF // nlanes

    mesh = plsc.VectorSubcoreMesh(core_axis_name="c", subcore_axis_name="s")

    @functools.partial(
        pl.kernel,
        mesh=mesh,
        compiler_params=pltpu.CompilerParams(
            needs_layout_passes=False, use_tc_tiling_on_sc=False),
        out_type=jax.ShapeDtypeStruct((N_NODES, IDENT_DIM), jnp.float32),
        scratch_types=[
            pltpu.VMEM((ipw,), jnp.int32),
            pltpu.VMEM((NBUF, rows, HALF), jnp.int32),
            pltpu.VMEM((out_rows, IDENT_DIM), jnp.float32),
        ]
        + [pltpu.SemaphoreType.DMA for _ in range(2 * NBUF)],
    )
    def body(idx_hbm, table_hbm, out_hbm, idx_v, ring_v, out_v, *sems):
        wid = lax.axis_index("s") * nc + lax.axis_index("c")
        nbase = wid * npw
        pltpu.sync_copy(idx_hbm.at[pl.ds(wid * ipw, ipw)], idx_v)

        split = 104

        def start(g, b):
            off = g * rows
            pltpu.async_copy(
                table_hbm.at[idx_v.at[pl.ds(off, split)]],
                ring_v.at[b, pl.ds(0, split)], sems[2 * b])
            pltpu.async_copy(
                table_hbm.at[idx_v.at[pl.ds(off + split, rows - split)]],
                ring_v.at[b, pl.ds(split, rows - split)], sems[2 * b + 1])

        def wait(b):
            pltpu.make_async_copy(
                table_hbm.at[idx_v.at[pl.ds(0, split)]],
                ring_v.at[b, pl.ds(0, split)], sems[2 * b]).wait()
            pltpu.make_async_copy(
                table_hbm.at[idx_v.at[pl.ds(0, rows - split)]],
                ring_v.at[b, pl.ds(split, rows - split)], sems[2 * b + 1]).wait()

        for b in range(NBUF):
            start(b, b)

        hi_mask = jnp.int32(-65536)

        def outer(gg, carry):
            g0 = gg * NBUF
            kk = gg // (FLUSH_GROUPS // NBUF)
            for b in range(NBUF):
                g = g0 + b
                wait(b)
                for h in range(GROUP):
                    row0 = h * EXPR_LEN
                    orow = (g - kk * FLUSH_GROUPS) * GROUP + h

                    zero = jnp.zeros((nlanes,), jnp.float32)

                    @plsc.parallel_loop(
                        0, EXPR_LEN, unroll=2,
                        carry=(zero,) * (2 * nchunks))
                    def accs(r, acc_in):
                        out = []
                        for c in range(nchunks):
                            w = ring_v[b, row0 + r, pl.ds(16 * c, 16)]
                            lo = lax.bitcast_convert_type(
                                lax.shift_left(w, 16), jnp.float32)
                            hi = lax.bitcast_convert_type(w, jnp.float32)
                            out.append(acc_in[2 * c] + lo)
                            out.append(acc_in[2 * c + 1] + hi)
                        return tuple(out)
                    for c in range(nchunks):
                        out_v[orow, pl.ds(16 * c, 16)] = accs[2 * c]
                        out_v[orow, pl.ds(HALF + 16 * c, 16)] = accs[2 * c + 1]

                    if h == GROUP - 1:
                        @pl.when(g + NBUF < ngroups)
                        def _start_next():
                            start(g + NBUF, b)

            @pl.when(gg % (FLUSH_GROUPS // NBUF) == FLUSH_GROUPS // NBUF - 1)
            def _flush():
                pltpu.sync_copy(
                    out_v,
                    out_hbm.at[pl.ds(nbase + kk * out_rows, out_rows)])

            return carry

        lax.fori_loop(0, ngroups // NBUF, outer, 0, unroll=False)

    return body(flat_idx, packed_table)


def _tc_project(pooled_sum, W_expr, b_expr, control_kind, control_kind_table):
    bn = 2048
    grid = (N_NODES // bn,)

    def body(x_ref, w_ref, b_ref, ck_ref, ctab_ref, o_ref):
        x = x_ref[...] * (1.0 / EXPR_LEN)
        y = jnp.dot(x, w_ref[...], preferred_element_type=jnp.float32)
        y = y + b_ref[...]
        ck = ck_ref[...]
        onehot = (
            ck == lax.broadcasted_iota(jnp.int32, (bn, CONTROL_VOCAB), 1)
        ).astype(jnp.float32)
        ctl = jnp.dot(onehot, ctab_ref[...], preferred_element_type=jnp.float32)
        o_ref[...] = jnp.concatenate([y, ctl], axis=-1)

    return pl.pallas_call(
        body,
        grid=grid,
        in_specs=[
            pl.BlockSpec((bn, IDENT_DIM), lambda i: (i, 0)),
            pl.BlockSpec((IDENT_DIM, IDENT_DIM), lambda i: (0, 0)),
            pl.BlockSpec((1, IDENT_DIM), lambda i: (0, 0)),
            pl.BlockSpec((bn, 1), lambda i: (i, 0)),
            pl.BlockSpec((CONTROL_VOCAB, CONTROL_DIM), lambda i: (0, 0)),
        ],
        out_specs=pl.BlockSpec((bn, OUT_DIM), lambda i: (i, 0)),
        out_shape=jax.ShapeDtypeStruct((N_NODES, OUT_DIM), jnp.float32),
    )(pooled_sum, W_expr, b_expr, control_kind, control_kind_table)


def kernel(encoded_identifiers, cfg_nodes_expressions, cfg_nodes_control_kind,
           W_expr, b_expr, control_kind_table):
    flat_idx = cfg_nodes_expressions.astype(jnp.int32).reshape(-1)
    ck = cfg_nodes_control_kind.astype(jnp.int32).reshape(N_NODES, 1)

    packed = _sc_pack_table(encoded_identifiers)
    pooled_sum = _sc_pool_sum(flat_idx, packed)
    return _tc_project(
        pooled_sum,
        W_expr,
        b_expr.reshape(1, IDENT_DIM),
        ck,
        control_kind_table,
    )
